# Initial kernel scaffold; baseline (speedup 1.0000x reference)
import jax
import jax.numpy as jnp
from jax.experimental import pallas as pl


def kernel(index, output, label, Q):
    raise NotImplementedError("write your pallas kernel here")


# speedup vs baseline: 9.5046x; 9.5046x over previous
import functools

import jax
import jax.numpy as jnp
from jax import lax
from jax.experimental import pallas as pl
from jax.experimental.pallas import tpu as pltpu
from jax.experimental.pallas import tpu_sc as plsc

EMA = 0.7
LAM = 3.0
CLIP_LO = 0.0001
CLIP_HI = 1.0 - 0.0001

NC = 2
NS = 16
NW = NC * NS
CH = 128
BS = 512


def _stats_body(out_ref, lab_ref, ce_ref, sp_ref, ydet_ref):
    x = out_ref[...]
    lab = lab_ref[0, 0, :]
    m = jnp.max(x, axis=1, keepdims=True)
    ex = jnp.exp(x - m)
    s = jnp.sum(ex, axis=1, keepdims=True)
    cols = lax.broadcasted_iota(jnp.int32, x.shape, 1)
    picked = jnp.sum(jnp.where(cols == lab[:, None], x, 0.0), axis=1)
    ce_ref[0, 0, :] = jnp.log(s[:, 0]) + m[:, 0] - picked
    yp = jnp.clip(ex / s, CLIP_LO, CLIP_HI)
    sp = jnp.sum(yp, axis=1)
    sp_ref[0, 0, :] = sp
    ydet_ref[...] = yp / sp[:, None]


def _tc_stats(output, label3):
    B, C = output.shape
    G = B // BS
    return pl.pallas_call(
        _stats_body,
        grid=(G,),
        in_specs=[
            pl.BlockSpec((BS, C), lambda i: (i, 0)),
            pl.BlockSpec((1, 1, BS), lambda i: (i, 0, 0)),
        ],
        out_specs=[
            pl.BlockSpec((1, 1, BS), lambda i: (i, 0, 0)),
            pl.BlockSpec((1, 1, BS), lambda i: (i, 0, 0)),
            pl.BlockSpec((BS, C), lambda i: (i, 0)),
        ],
        out_shape=[
            jax.ShapeDtypeStruct((G, 1, BS), jnp.float32),
            jax.ShapeDtypeStruct((G, 1, BS), jnp.float32),
            jax.ShapeDtypeStruct((B, C), jnp.float32),
        ],
    )(output, label3)


def _sc_a_body(B, idx_hbm, rid_hbm, q_hbm, t_hbm, qrows_hbm,
               idx_v, rid_v, rows_v, sem):
    c = lax.axis_index("c")
    s = lax.axis_index("s")
    wid = s * NC + c
    rpw = B // NW

    @pl.when(wid == 0)
    def _scatter():
        def step(k, carry):
            pltpu.sync_copy(idx_hbm.at[pl.ds(k * CH, CH)], idx_v)
            pltpu.sync_copy(rid_hbm.at[pl.ds(k * CH, CH)], rid_v)
            pltpu.async_copy(rid_v, t_hbm.at[idx_v], sem).wait()
            return carry
        lax.fori_loop(0, B // CH, step, 0)

    def qstep(k, carry):
        base = wid * rpw + k * CH
        pltpu.sync_copy(idx_hbm.at[pl.ds(base, CH)], idx_v)
        pltpu.async_copy(q_hbm.at[idx_v], rows_v, sem).wait()
        pltpu.sync_copy(rows_v, qrows_hbm.at[pl.ds(base, CH)])
        return carry
    lax.fori_loop(0, rpw // CH, qstep, 0)


def _sc_a(index, rowid, Q):
    B = index.shape[0]
    N, C = Q.shape
    mesh = plsc.VectorSubcoreMesh(core_axis_name="c", subcore_axis_name="s",
                                  num_cores=NC, num_subcores=NS)
    return pl.kernel(
        functools.partial(_sc_a_body, B),
        out_type=[
            jax.ShapeDtypeStruct((N,), jnp.int32),
            jax.ShapeDtypeStruct((B, C), jnp.float32),
        ],
        mesh=mesh,
        scratch_types=[
            pltpu.VMEM((CH,), jnp.int32),
            pltpu.VMEM((CH,), jnp.int32),
            pltpu.VMEM((CH, C), jnp.float32),
            pltpu.SemaphoreType.DMA,
        ],
    )(index, rowid, Q)


def _sc_b_body(B, idx_hbm, t_hbm, ydet_hbm, ydl_hbm, idx_v, jl_v, rows_v, sem):
    c = lax.axis_index("c")
    s = lax.axis_index("s")
    wid = s * NC + c
    rpw = B // NW

    def dstep(k, carry):
        base = wid * rpw + k * CH
        pltpu.sync_copy(idx_hbm.at[pl.ds(base, CH)], idx_v)
        pltpu.async_copy(t_hbm.at[idx_v], jl_v, sem).wait()
        pltpu.async_copy(ydet_hbm.at[jl_v], rows_v, sem).wait()
        pltpu.sync_copy(rows_v, ydl_hbm.at[pl.ds(base, CH)])
        return carry
    lax.fori_loop(0, rpw // CH, dstep, 0)


def _sc_b(index, t, ydet):
    B, C = ydet.shape
    mesh = plsc.VectorSubcoreMesh(core_axis_name="c", subcore_axis_name="s",
                                  num_cores=NC, num_subcores=NS)
    return pl.kernel(
        functools.partial(_sc_b_body, B),
        out_type=jax.ShapeDtypeStruct((B, C), jnp.float32),
        mesh=mesh,
        scratch_types=[
            pltpu.VMEM((CH,), jnp.int32),
            pltpu.VMEM((CH,), jnp.int32),
            pltpu.VMEM((CH, C), jnp.float32),
            pltpu.SemaphoreType.DMA,
        ],
    )(index, t, ydet)


def _loss_body(B, qrows_ref, ydl_ref, ydet_ref, ce_ref, sp_ref, out_ref):
    g = EMA * qrows_ref[...] + (1.0 - EMA) * ydl_ref[...]
    ydet = ydet_ref[...]
    inner = jnp.sum(g * ydet, axis=1) * sp_ref[0, 0, :]
    part = jnp.sum(ce_ref[0, 0, :] + LAM * jnp.log(1.0 - inner))

    @pl.when(pl.program_id(0) == 0)
    def _():
        out_ref[...] = jnp.zeros((1, 1), jnp.float32)
    out_ref[...] += part[None, None]

    @pl.when(pl.program_id(0) == pl.num_programs(0) - 1)
    def _():
        out_ref[...] = out_ref[...] / B


def _tc_loss(qrows, ydl, ydet, ce3, sp3):
    B, C = ydet.shape
    G = B // BS
    return pl.pallas_call(
        functools.partial(_loss_body, B),
        grid=(G,),
        in_specs=[
            pl.BlockSpec((BS, C), lambda i: (i, 0)),
            pl.BlockSpec((BS, C), lambda i: (i, 0)),
            pl.BlockSpec((BS, C), lambda i: (i, 0)),
            pl.BlockSpec((1, 1, BS), lambda i: (i, 0, 0)),
            pl.BlockSpec((1, 1, BS), lambda i: (i, 0, 0)),
        ],
        out_specs=pl.BlockSpec((1, 1), lambda i: (0, 0)),
        out_shape=jax.ShapeDtypeStruct((1, 1), jnp.float32),
    )(qrows, ydl, ydet, ce3, sp3)


def kernel(index, output, label, Q):
    B, C = output.shape
    label3 = label.astype(jnp.int32).reshape(B // BS, 1, BS)
    rowid = jnp.arange(B, dtype=jnp.int32)

    ce3, sp3, ydet = _tc_stats(output, label3)
    t, qrows = _sc_a(index, rowid, Q)
    ydl = _sc_b(index, t, ydet)
    out = _tc_loss(qrows, ydl, ydet, ce3, sp3)
    return out[0, 0]


# speedup vs baseline: 23.6101x; 2.4841x over previous
import functools

import jax
import jax.numpy as jnp
from jax import lax
from jax.experimental import pallas as pl
from jax.experimental.pallas import tpu as pltpu
from jax.experimental.pallas import tpu_sc as plsc

EMA = 0.7
LAM = 3.0
CLIP_LO = 0.0001
CLIP_HI = 1.0 - 0.0001

NC = 2
NS = 16
NW = NC * NS
L = 16
CH = 128
BS = 512


def _wid():
    return lax.axis_index("s") * NC + lax.axis_index("c")


def _stats_body(out_ref, lab_ref, ce_ref, sp_ref, ydet_ref):
    x = out_ref[...]
    lab = lab_ref[0, 0, :]
    m = jnp.max(x, axis=1, keepdims=True)
    ex = jnp.exp(x - m)
    s = jnp.sum(ex, axis=1, keepdims=True)
    cols = lax.broadcasted_iota(jnp.int32, x.shape, 1)
    picked = jnp.sum(jnp.where(cols == lab[:, None], x, 0.0), axis=1)
    ce_ref[0, 0, :] = jnp.log(s[:, 0]) + m[:, 0] - picked
    yp = jnp.clip(ex / s, CLIP_LO, CLIP_HI)
    sp = jnp.sum(yp, axis=1)
    sp_ref[0, 0, :] = sp
    ydet_ref[...] = yp / sp[:, None]


def _tc_stats(output, label3):
    B, C = output.shape
    G = B // BS
    return pl.pallas_call(
        _stats_body,
        grid=(G,),
        in_specs=[
            pl.BlockSpec((BS, C), lambda i: (i, 0)),
            pl.BlockSpec((1, 1, BS), lambda i: (i, 0, 0)),
        ],
        out_specs=[
            pl.BlockSpec((1, 1, BS), lambda i: (i, 0, 0)),
            pl.BlockSpec((1, 1, BS), lambda i: (i, 0, 0)),
            pl.BlockSpec((BS, C), lambda i: (i, 0)),
        ],
        out_shape=[
            jax.ShapeDtypeStruct((G, 1, BS), jnp.float32),
            jax.ShapeDtypeStruct((G, 1, BS), jnp.float32),
            jax.ShapeDtypeStruct((B, C), jnp.float32),
        ],
    )(output, label3)


def _sc_a_body(B, vsp, idx_hbm, q_hbm, t_hbm, qrows_hbm,
               idx_all, t_loc, rows_v, sem, sem2):
    wid = _wid()
    rpw = B // NW
    lo = wid * vsp

    pltpu.sync_copy(idx_hbm, idx_all)

    qc = []
    for k in range(rpw // CH):
        base = wid * rpw + k * CH
        qc.append(pltpu.async_copy(
            q_hbm.at[idx_all.at[pl.ds(base, CH)]],
            rows_v.at[pl.ds(k * CH, CH)], sem))

    def step(k, carry):
        iv = idx_all[pl.ds(k * L, L)]
        jv = k * L + lax.iota(jnp.int32, L)
        owned = (iv >= lo) & (iv < lo + vsp)
        locv = jnp.clip(iv - lo, 0, vsp - 1)
        plsc.store_scatter(t_loc, [locv], jv, mask=owned)
        return carry
    lax.fori_loop(0, B // L, step, 0)
    pltpu.sync_copy(t_loc, t_hbm.at[pl.ds(lo, vsp)])

    for cp in qc:
        cp.wait()
    pltpu.sync_copy(rows_v, qrows_hbm.at[pl.ds(wid * rpw, rpw)])


def _sc_a(index, Q, vsp):
    B = index.shape[0]
    N, C = Q.shape
    rpw = B // NW
    mesh = plsc.VectorSubcoreMesh(core_axis_name="c", subcore_axis_name="s",
                                  num_cores=NC, num_subcores=NS)
    return pl.kernel(
        functools.partial(_sc_a_body, B, vsp),
        out_type=[
            jax.ShapeDtypeStruct((NW * vsp,), jnp.int32),
            jax.ShapeDtypeStruct((B, C), jnp.float32),
        ],
        mesh=mesh,
        compiler_params=pltpu.CompilerParams(needs_layout_passes=False),
        scratch_types=[
            pltpu.VMEM((B,), jnp.int32),
            pltpu.VMEM((vsp,), jnp.int32),
            pltpu.VMEM((rpw, C), jnp.float32),
            pltpu.SemaphoreType.DMA,
            pltpu.SemaphoreType.DMA,
        ],
    )(index, Q)


def _sc_b_body(B, idx3_hbm, t_hbm, ydet_hbm, ydl_hbm, idx_v, jl_v, rows_v,
               sem, sem2):
    wid = _wid()
    rpw = B // NW
    nk = rpw // CH

    pltpu.sync_copy(idx3_hbm.at[wid], idx_v)
    jc = [pltpu.async_copy(t_hbm.at[idx_v.at[k]], jl_v.at[k], sem2)
          for k in range(nk)]
    for cp in jc:
        cp.wait()
    dc = [pltpu.async_copy(ydet_hbm.at[jl_v.at[k]],
                           rows_v.at[pl.ds(k * CH, CH)], sem)
          for k in range(nk)]
    for cp in dc:
        cp.wait()
    pltpu.sync_copy(rows_v, ydl_hbm.at[pl.ds(wid * rpw, rpw)])


def _sc_b(index3, t, ydet):
    B, C = ydet.shape
    rpw = B // NW
    mesh = plsc.VectorSubcoreMesh(core_axis_name="c", subcore_axis_name="s",
                                  num_cores=NC, num_subcores=NS)
    return pl.kernel(
        functools.partial(_sc_b_body, B),
        out_type=jax.ShapeDtypeStruct((B, C), jnp.float32),
        mesh=mesh,
        scratch_types=[
            pltpu.VMEM((rpw // CH, CH), jnp.int32),
            pltpu.VMEM((rpw // CH, CH), jnp.int32),
            pltpu.VMEM((rpw, C), jnp.float32),
            pltpu.SemaphoreType.DMA,
            pltpu.SemaphoreType.DMA,
        ],
    )(index3, t, ydet)


def _loss_body(B, qrows_ref, ydl_ref, ydet_ref, ce_ref, sp_ref, out_ref):
    g = EMA * qrows_ref[...] + (1.0 - EMA) * ydl_ref[...]
    ydet = ydet_ref[...]
    inner = jnp.sum(g * ydet, axis=1) * sp_ref[0, 0, :]
    part = jnp.sum(ce_ref[0, 0, :] + LAM * jnp.log(1.0 - inner))

    @pl.when(pl.program_id(0) == 0)
    def _():
        out_ref[...] = jnp.zeros((1, 1), jnp.float32)
    out_ref[...] += part[None, None]

    @pl.when(pl.program_id(0) == pl.num_programs(0) - 1)
    def _():
        out_ref[...] = out_ref[...] / B


def _tc_loss(qrows, ydl, ydet, ce3, sp3):
    B, C = ydet.shape
    G = B // BS
    return pl.pallas_call(
        functools.partial(_loss_body, B),
        grid=(G,),
        in_specs=[
            pl.BlockSpec((BS, C), lambda i: (i, 0)),
            pl.BlockSpec((BS, C), lambda i: (i, 0)),
            pl.BlockSpec((BS, C), lambda i: (i, 0)),
            pl.BlockSpec((1, 1, BS), lambda i: (i, 0, 0)),
            pl.BlockSpec((1, 1, BS), lambda i: (i, 0, 0)),
        ],
        out_specs=pl.BlockSpec((1, 1), lambda i: (0, 0)),
        out_shape=jax.ShapeDtypeStruct((1, 1), jnp.float32),
    )(qrows, ydl, ydet, ce3, sp3)


def kernel(index, output, label, Q):
    B, C = output.shape
    N = Q.shape[0]
    rpw = B // NW
    vsp = ((N + NW - 1) // NW + 7) // 8 * 8

    label3 = label.astype(jnp.int32).reshape(B // BS, 1, BS)
    idx = index.astype(jnp.int32)
    index3 = idx.reshape(NW, rpw // CH, CH)

    ce3, sp3, ydet = _tc_stats(output, label3)
    t, qrows = _sc_a(idx, Q, vsp)
    ydl = _sc_b(index3, t, ydet)
    out = _tc_loss(qrows, ydl, ydet, ce3, sp3)
    return out[0, 0]
